# initial kernel scaffold (unmeasured)
import jax
import jax.numpy as jnp
from jax import lax
from jax.experimental import pallas as pl
from jax.experimental.pallas import tpu as pltpu

T = 2048
D = 4096
V_SHARD = 8192
V_TILE = 512
N_TILES = V_SHARD // V_TILE


def _stats_body(x_ref, w_ref, labels_ref, stats_ref):
    i = pl.program_id(0)
    my_x = lax.axis_index("x")

    logits = jnp.dot(x_ref[:, :], w_ref[:, :], preferred_element_type=jnp.float32)
    sumexp = jnp.sum(jnp.exp(logits), axis=1, keepdims=True)
    col0 = my_x * V_SHARD + i * V_TILE
    cols = lax.broadcasted_iota(jnp.int32, logits.shape, 1) + col0
    lab = jnp.sum(
        jnp.where(cols == labels_ref[:, :], logits, 0.0), axis=1, keepdims=True
    )

    @pl.when(i == 0)
    def _():
        stats_ref[:, 0:1] = sumexp
        stats_ref[:, 1:2] = lab

    @pl.when(i != 0)
    def _():
        stats_ref[:, 0:1] += sumexp
        stats_ref[:, 1:2] += lab


def _combine_body(stats_ref, out_ref, comm_ref, send_sem, recv_sem):
    my_x = lax.axis_index("x")
    my_y = lax.axis_index("y")
    peer = (1 - my_x, my_y)

    barrier = pltpu.get_barrier_semaphore()
    pl.semaphore_signal(
        barrier, inc=1, device_id=peer, device_id_type=pl.DeviceIdType.MESH
    )
    pl.semaphore_wait(barrier, 1)

    rdma = pltpu.make_async_remote_copy(
        src_ref=stats_ref,
        dst_ref=comm_ref,
        send_sem=send_sem,
        recv_sem=recv_sem,
        device_id=peer,
        device_id_type=pl.DeviceIdType.MESH,
    )
    rdma.start()
    rdma.wait()

    s = stats_ref[:, 0:1] + comm_ref[:, 0:1]
    lab = stats_ref[:, 1:2] + comm_ref[:, 1:2]
    out_ref[:, :] = jnp.log(s) - lab


def kernel(x, W, labels):
    labels2 = labels.reshape(T, 1).astype(jnp.int32)

    stats = pl.pallas_call(
        _stats_body,
        grid=(N_TILES,),
        in_specs=[
            pl.BlockSpec((T, D), lambda i: (0, 0)),
            pl.BlockSpec((D, V_TILE), lambda i: (0, i)),
            pl.BlockSpec((T, 1), lambda i: (0, 0)),
        ],
        out_specs=pl.BlockSpec((T, 2), lambda i: (0, 0)),
        out_shape=jax.ShapeDtypeStruct((T, 2), jnp.float32),
        compiler_params=pltpu.CompilerParams(
            dimension_semantics=("arbitrary",),
        ),
    )(x, W, labels2)

    nll = pl.pallas_call(
        _combine_body,
        in_specs=[pl.BlockSpec(memory_space=pltpu.VMEM)],
        out_specs=pl.BlockSpec(memory_space=pltpu.VMEM),
        out_shape=jax.ShapeDtypeStruct((T, 1), jnp.float32),
        scratch_shapes=[
            pltpu.VMEM((T, 2), jnp.float32),
            pltpu.SemaphoreType.DMA,
            pltpu.SemaphoreType.DMA,
        ],
        compiler_params=pltpu.CompilerParams(collective_id=0),
    )(stats)

    return nll[:, 0]


# baseline (device time: 189669 ns/iter reference)
import jax
import jax.numpy as jnp
from jax import lax
from jax.experimental import pallas as pl
from jax.experimental.pallas import tpu as pltpu

T = 2048
D = 4096
V_SHARD = 8192
V_TILE = 512
N_TILES = V_SHARD // V_TILE


def _stats_body(x_ref, w_ref, labels_ref, stats_ref):
    i = pl.program_id(0)
    my_x = lax.axis_index("x")

    logits = jnp.dot(x_ref[:, :], w_ref[:, :], preferred_element_type=jnp.float32)
    sumexp = jnp.sum(jnp.exp(logits), axis=1, keepdims=True)
    col0 = my_x * V_SHARD + i * V_TILE
    cols = lax.broadcasted_iota(jnp.int32, logits.shape, 1) + col0
    lab = jnp.sum(
        jnp.where(cols == labels_ref[:, :], logits, 0.0), axis=1, keepdims=True
    )

    @pl.when(i == 0)
    def _():
        stats_ref[:, 0:1] = sumexp
        stats_ref[:, 1:2] = lab

    @pl.when(i != 0)
    def _():
        stats_ref[:, 0:1] += sumexp
        stats_ref[:, 1:2] += lab


def _combine_body(stats_ref, out_ref, comm_ref, send_sem, recv_sem):
    my_x = lax.axis_index("x")
    my_y = lax.axis_index("y")
    peer = (1 - my_x, my_y)

    barrier = pltpu.get_barrier_semaphore()
    pl.semaphore_signal(
        barrier, inc=1, device_id=peer, device_id_type=pl.DeviceIdType.MESH
    )
    pl.semaphore_wait(barrier, 1)

    rdma = pltpu.make_async_remote_copy(
        src_ref=stats_ref,
        dst_ref=comm_ref,
        send_sem=send_sem,
        recv_sem=recv_sem,
        device_id=peer,
        device_id_type=pl.DeviceIdType.MESH,
    )
    rdma.start()
    rdma.wait()

    s = stats_ref[:, 0:1] + comm_ref[:, 0:1]
    lab = stats_ref[:, 1:2] + comm_ref[:, 1:2]
    out_ref[:, :] = jnp.log(s) - lab


def kernel(x, W, labels):
    labels2 = labels.reshape(T, 1).astype(jnp.int32)

    stats = pl.pallas_call(
        _stats_body,
        grid=(N_TILES,),
        in_specs=[
            pl.BlockSpec((T, D), lambda i: (0, 0)),
            pl.BlockSpec((D, V_TILE), lambda i: (0, i)),
            pl.BlockSpec((T, 1), lambda i: (0, 0)),
        ],
        out_specs=pl.BlockSpec((T, 2), lambda i: (0, 0)),
        out_shape=jax.ShapeDtypeStruct((T, 2), jnp.float32),
        compiler_params=pltpu.CompilerParams(
            dimension_semantics=("arbitrary",),
            vmem_limit_bytes=100 * 1024 * 1024,
        ),
    )(x, W, labels2)

    nll = pl.pallas_call(
        _combine_body,
        in_specs=[pl.BlockSpec(memory_space=pltpu.VMEM)],
        out_specs=pl.BlockSpec(memory_space=pltpu.VMEM),
        out_shape=jax.ShapeDtypeStruct((T, 1), jnp.float32),
        scratch_shapes=[
            pltpu.VMEM((T, 2), jnp.float32),
            pltpu.SemaphoreType.DMA,
            pltpu.SemaphoreType.DMA,
        ],
        compiler_params=pltpu.CompilerParams(collective_id=0),
    )(stats)

    return nll[:, 0]


# device time: 117642 ns/iter; 1.6123x vs baseline; 1.6123x over previous
import jax
import jax.numpy as jnp
from jax import lax
from jax.experimental import pallas as pl
from jax.experimental.pallas import tpu as pltpu

T = 2048
H = T // 2
D = 4096
V_SHARD = 8192
V_TILE = 512
N_TILES = V_SHARD // V_TILE


def _stats_body(x_ref, w_ref, labels_ref, stats_ref):
    i = pl.program_id(0)
    my_x = lax.axis_index("x")
    my_y = lax.axis_index("y")
    r0 = my_y * H

    xh = x_ref[pl.ds(r0, H), :]
    logits = jnp.dot(xh, w_ref[:, :], preferred_element_type=jnp.float32)
    sumexp = jnp.sum(jnp.exp(logits), axis=1, keepdims=True)
    col0 = my_x * V_SHARD + i * V_TILE
    cols = lax.broadcasted_iota(jnp.int32, logits.shape, 1) + col0
    labs = labels_ref[pl.ds(r0, H), :]
    lab = jnp.sum(
        jnp.where(cols == labs, logits, 0.0), axis=1, keepdims=True
    )

    @pl.when(i == 0)
    def _():
        stats_ref[:, 0:1] = sumexp
        stats_ref[:, 1:2] = lab

    @pl.when(i != 0)
    def _():
        stats_ref[:, 0:1] += sumexp
        stats_ref[:, 1:2] += lab


def _combine_body(stats_ref, out_ref, comm_ref, s1_send, s1_recv, s2_send, s2_recv):
    my_x = lax.axis_index("x")
    my_y = lax.axis_index("y")
    xpeer = (1 - my_x, my_y)
    ypeer = (my_x, 1 - my_y)

    barrier = pltpu.get_barrier_semaphore()
    pl.semaphore_signal(
        barrier, inc=1, device_id=xpeer, device_id_type=pl.DeviceIdType.MESH
    )
    pl.semaphore_signal(
        barrier, inc=1, device_id=ypeer, device_id_type=pl.DeviceIdType.MESH
    )
    pl.semaphore_wait(barrier, 2)

    rdma1 = pltpu.make_async_remote_copy(
        src_ref=stats_ref,
        dst_ref=comm_ref,
        send_sem=s1_send,
        recv_sem=s1_recv,
        device_id=xpeer,
        device_id_type=pl.DeviceIdType.MESH,
    )
    rdma1.start()
    rdma1.wait()

    s = stats_ref[:, 0:1] + comm_ref[:, 0:1]
    lab = stats_ref[:, 1:2] + comm_ref[:, 1:2]
    out_ref[pl.ds(my_y * H, H), :] = jnp.log(s) - lab

    rdma2 = pltpu.make_async_remote_copy(
        src_ref=out_ref.at[pl.ds(my_y * H, H), :],
        dst_ref=out_ref.at[pl.ds(my_y * H, H), :],
        send_sem=s2_send,
        recv_sem=s2_recv,
        device_id=ypeer,
        device_id_type=pl.DeviceIdType.MESH,
    )
    rdma2.start()
    rdma2.wait()


def kernel(x, W, labels):
    labels2 = labels.reshape(T, 1).astype(jnp.int32)

    stats = pl.pallas_call(
        _stats_body,
        grid=(N_TILES,),
        in_specs=[
            pl.BlockSpec((T, D), lambda i: (0, 0)),
            pl.BlockSpec((D, V_TILE), lambda i: (0, i)),
            pl.BlockSpec((T, 1), lambda i: (0, 0)),
        ],
        out_specs=pl.BlockSpec((H, 2), lambda i: (0, 0)),
        out_shape=jax.ShapeDtypeStruct((H, 2), jnp.float32),
        compiler_params=pltpu.CompilerParams(
            dimension_semantics=("arbitrary",),
            vmem_limit_bytes=100 * 1024 * 1024,
        ),
    )(x, W, labels2)

    nll = pl.pallas_call(
        _combine_body,
        in_specs=[pl.BlockSpec(memory_space=pltpu.VMEM)],
        out_specs=pl.BlockSpec(memory_space=pltpu.VMEM),
        out_shape=jax.ShapeDtypeStruct((T, 1), jnp.float32),
        scratch_shapes=[
            pltpu.VMEM((H, 2), jnp.float32),
            pltpu.SemaphoreType.DMA,
            pltpu.SemaphoreType.DMA,
            pltpu.SemaphoreType.DMA,
            pltpu.SemaphoreType.DMA,
        ],
        compiler_params=pltpu.CompilerParams(collective_id=0),
    )(stats)

    return nll[:, 0]
